# baseline (speedup 1.0000x reference)
import jax
import jax.numpy as jnp
from jax.experimental import pallas as pl
from jax.experimental.pallas import tpu as pltpu

B = 1024
D = 128
N = 100000
TILE_N = 2000
TEMP = 0.05
CAP = 1.0 / TEMP


def _cm_kernel(x_rgb_ref, x_ir_ref, t_rgb_ref, t_ir_ref, f_rgb_ref, f_ir_ref,
               out_rgb_ref, out_ir_ref,
               xn_rgb, xn_ir, s_rgb, s_ir, p_rgb, p_ir):
    c = pl.program_id(0)
    nc = pl.num_programs(0)

    @pl.when(c == 0)
    def _init():
        for x_ref, xn in ((x_rgb_ref, xn_rgb), (x_ir_ref, xn_ir)):
            x = x_ref[...]
            n = jnp.sqrt(jnp.sum(x * x, axis=1, keepdims=True))
            xn[...] = x / jnp.maximum(n, 1e-12)
        s_rgb[...] = jnp.zeros_like(s_rgb)
        s_ir[...] = jnp.zeros_like(s_ir)
        p_rgb[...] = jnp.zeros_like(p_rgb)
        p_ir[...] = jnp.zeros_like(p_ir)

    col = c * TILE_N + jax.lax.broadcasted_iota(jnp.int32, (1, TILE_N), 1)

    def bank(xn, t_ref, f_ref, s_acc, p_acc):
        logits = jax.lax.dot_general(
            xn[...].astype(jnp.bfloat16), f_ref[...].astype(jnp.bfloat16),
            (((1,), (1,)), ((), ())),
            preferred_element_type=jnp.float32) * (1.0 / TEMP)
        s_acc[...] += jnp.sum(jnp.exp(logits - CAP), axis=1, keepdims=True)
        mask = t_ref[...] == col
        p_acc[...] += jnp.sum(jnp.where(mask, logits, 0.0), axis=1,
                              keepdims=True)

    bank(xn_rgb, t_rgb_ref, f_rgb_ref, s_rgb, p_rgb)
    bank(xn_ir, t_ir_ref, f_ir_ref, s_ir, p_ir)

    @pl.when(c == nc - 1)
    def _fin():
        out_rgb_ref[...] = jnp.mean(
            CAP + jnp.log(s_rgb[...]) - p_rgb[...]).reshape(1, 1)
        out_ir_ref[...] = jnp.mean(
            CAP + jnp.log(s_ir[...]) - p_ir[...]).reshape(1, 1)


@jax.jit
def _run(x_rgb, x_ir, t_rgb, t_ir, f_rgb, f_ir):
    out = pl.pallas_call(
        _cm_kernel,
        grid=(N // TILE_N,),
        in_specs=[
            pl.BlockSpec((B, D), lambda c: (0, 0)),
            pl.BlockSpec((B, D), lambda c: (0, 0)),
            pl.BlockSpec((B, 1), lambda c: (0, 0)),
            pl.BlockSpec((B, 1), lambda c: (0, 0)),
            pl.BlockSpec((TILE_N, D), lambda c: (c, 0)),
            pl.BlockSpec((TILE_N, D), lambda c: (c, 0)),
        ],
        out_specs=[
            pl.BlockSpec((1, 1), lambda c: (0, 0)),
            pl.BlockSpec((1, 1), lambda c: (0, 0)),
        ],
        out_shape=[jax.ShapeDtypeStruct((1, 1), jnp.float32)] * 2,
        scratch_shapes=[
            pltpu.VMEM((B, D), jnp.float32),
            pltpu.VMEM((B, D), jnp.float32),
            pltpu.VMEM((B, 1), jnp.float32),
            pltpu.VMEM((B, 1), jnp.float32),
            pltpu.VMEM((B, 1), jnp.float32),
            pltpu.VMEM((B, 1), jnp.float32),
        ],
        compiler_params=pltpu.CompilerParams(
            dimension_semantics=("arbitrary",)),
    )(x_rgb, x_ir, t_rgb, t_ir, f_rgb, f_ir)
    return out[0][0, 0], out[1][0, 0]


def kernel(inputs_rgb, inputs_ir, targets_rgb, targets_ir,
           features_rgb, features_ir):
    return _run(inputs_rgb, inputs_ir,
                targets_rgb[:, None], targets_ir[:, None],
                features_rgb, features_ir)


# speedup vs baseline: 1.2453x; 1.2453x over previous
import math

import jax
import jax.numpy as jnp
from jax.experimental import pallas as pl
from jax.experimental.pallas import tpu as pltpu

B = 1024
D = 128
N = 100000
TILE_N = 2000
TEMP = 0.05
CAP = 1.0 / TEMP
LOG2E = math.log2(math.e)
QSCALE = LOG2E / TEMP
QCAP = CAP * LOG2E


def _cm_kernel(x_rgb_ref, x_ir_ref, t_rgb_ref, t_ir_ref, f_rgb_ref, f_ir_ref,
               out_rgb_ref, out_ir_ref,
               xn_rgb, xn_ir, s_rgb, s_ir, p_rgb, p_ir):
    c = pl.program_id(0)
    nc = pl.num_programs(0)

    @pl.when(c == 0)
    def _init():
        for x_ref, xn in ((x_rgb_ref, xn_rgb), (x_ir_ref, xn_ir)):
            x = x_ref[...]
            n = jnp.sqrt(jnp.sum(x * x, axis=1, keepdims=True))
            xn[...] = (x * (QSCALE / jnp.maximum(n, 1e-12))).astype(
                jnp.bfloat16)
        s_rgb[...] = jnp.zeros_like(s_rgb)
        s_ir[...] = jnp.zeros_like(s_ir)
        p_rgb[...] = jnp.zeros_like(p_rgb)
        p_ir[...] = jnp.zeros_like(p_ir)

    col = c * TILE_N + jax.lax.broadcasted_iota(jnp.int32, (1, TILE_N), 1)

    def bank(xn, t_ref, f_ref, s_acc, p_acc):
        q = jax.lax.dot_general(
            xn[...], f_ref[...].astype(jnp.bfloat16),
            (((1,), (1,)), ((), ())),
            preferred_element_type=jnp.float32)
        s_acc[...] += jnp.sum(jnp.exp2(q - QCAP), axis=1, keepdims=True)
        mask = t_ref[...] == col
        p_acc[...] += jnp.sum(jnp.where(mask, q, 0.0), axis=1, keepdims=True)

    bank(xn_rgb, t_rgb_ref, f_rgb_ref, s_rgb, p_rgb)
    bank(xn_ir, t_ir_ref, f_ir_ref, s_ir, p_ir)

    @pl.when(c == nc - 1)
    def _fin():
        out_rgb_ref[...] = jnp.mean(
            CAP + jnp.log(s_rgb[...]) - p_rgb[...] / LOG2E).reshape(1, 1)
        out_ir_ref[...] = jnp.mean(
            CAP + jnp.log(s_ir[...]) - p_ir[...] / LOG2E).reshape(1, 1)


@jax.jit
def _run(x_rgb, x_ir, t_rgb, t_ir, f_rgb, f_ir):
    out = pl.pallas_call(
        _cm_kernel,
        grid=(N // TILE_N,),
        in_specs=[
            pl.BlockSpec((B, D), lambda c: (0, 0)),
            pl.BlockSpec((B, D), lambda c: (0, 0)),
            pl.BlockSpec((B, 1), lambda c: (0, 0)),
            pl.BlockSpec((B, 1), lambda c: (0, 0)),
            pl.BlockSpec((TILE_N, D), lambda c: (c, 0)),
            pl.BlockSpec((TILE_N, D), lambda c: (c, 0)),
        ],
        out_specs=[
            pl.BlockSpec((1, 1), lambda c: (0, 0)),
            pl.BlockSpec((1, 1), lambda c: (0, 0)),
        ],
        out_shape=[jax.ShapeDtypeStruct((1, 1), jnp.float32)] * 2,
        scratch_shapes=[
            pltpu.VMEM((B, D), jnp.bfloat16),
            pltpu.VMEM((B, D), jnp.bfloat16),
            pltpu.VMEM((B, 1), jnp.float32),
            pltpu.VMEM((B, 1), jnp.float32),
            pltpu.VMEM((B, 1), jnp.float32),
            pltpu.VMEM((B, 1), jnp.float32),
        ],
        compiler_params=pltpu.CompilerParams(
            dimension_semantics=("arbitrary",)),
    )(x_rgb, x_ir, t_rgb, t_ir, f_rgb, f_ir)
    return out[0][0, 0], out[1][0, 0]


def kernel(inputs_rgb, inputs_ir, targets_rgb, targets_ir,
           features_rgb, features_ir):
    return _run(inputs_rgb, inputs_ir,
                targets_rgb[:, None], targets_ir[:, None],
                features_rgb, features_ir)


# speedup vs baseline: 1.7592x; 1.4126x over previous
import functools
import math

import jax
import jax.numpy as jnp
from jax.experimental import pallas as pl
from jax.experimental.pallas import tpu as pltpu
from jax.experimental.pallas import tpu_sc as plsc

B = 1024
D = 128
N = 100000
TILE_N = 2000
TEMP = 0.05
CAP = 1.0 / TEMP
LOG2E = math.log2(math.e)
QSCALE = LOG2E / TEMP
QCAP = CAP * LOG2E

_info = plsc.get_sparse_core_info()
_NW = _info.num_cores * _info.num_subcores
_BPW = B // _NW


def _gather_body(f_rgb_hbm, t_rgb_hbm, f_ir_hbm, t_ir_hbm,
                 g_rgb_hbm, g_ir_hbm, idx_v, rows_v, sem):
    wid = (jax.lax.axis_index("s") * _info.num_cores
           + jax.lax.axis_index("c"))
    base = wid * _BPW
    for f_hbm, t_hbm, g_hbm in ((f_rgb_hbm, t_rgb_hbm, g_rgb_hbm),
                                (f_ir_hbm, t_ir_hbm, g_ir_hbm)):
        pltpu.sync_copy(t_hbm.at[pl.ds(base, _BPW)], idx_v)
        pltpu.async_copy(f_hbm.at[idx_v], rows_v, sem).wait()
        pltpu.sync_copy(rows_v, g_hbm.at[pl.ds(base, _BPW)])


_sc_gather = pl.kernel(
    _gather_body,
    mesh=plsc.VectorSubcoreMesh(core_axis_name="c", subcore_axis_name="s"),
    out_type=[jax.ShapeDtypeStruct((B, D), jnp.float32)] * 2,
    scratch_types=[
        pltpu.VMEM((_BPW,), jnp.int32),
        pltpu.VMEM((_BPW, D), jnp.float32),
        pltpu.SemaphoreType.DMA,
    ],
)


def _cm_kernel(x_rgb_ref, x_ir_ref, g_rgb_ref, g_ir_ref,
               f_rgb_ref, f_ir_ref, out_rgb_ref, out_ir_ref,
               xn_rgb, xn_ir, inv_rgb, inv_ir, s_rgb, s_ir):
    c = pl.program_id(0)
    nc = pl.num_programs(0)

    @pl.when(c == 0)
    def _init():
        for x_ref, xn, inv in ((x_rgb_ref, xn_rgb, inv_rgb),
                               (x_ir_ref, xn_ir, inv_ir)):
            x = x_ref[...]
            n = jnp.sqrt(jnp.sum(x * x, axis=1, keepdims=True))
            r = 1.0 / jnp.maximum(n, 1e-12)
            xn[...] = (x * (QSCALE * r)).astype(jnp.bfloat16)
            inv[...] = r * (1.0 / TEMP)
        s_rgb[...] = jnp.zeros_like(s_rgb)
        s_ir[...] = jnp.zeros_like(s_ir)

    def bank(xn, f_ref, s_acc):
        q = jax.lax.dot_general(
            xn[...], f_ref[...].astype(jnp.bfloat16),
            (((1,), (1,)), ((), ())),
            preferred_element_type=jnp.float32)
        s_acc[...] += jnp.sum(jnp.exp2(q - QCAP), axis=1, keepdims=True)

    bank(xn_rgb, f_rgb_ref, s_rgb)
    bank(xn_ir, f_ir_ref, s_ir)

    @pl.when(c == nc - 1)
    def _fin():
        for x_ref, g_ref, inv, s_acc, out_ref in (
                (x_rgb_ref, g_rgb_ref, inv_rgb, s_rgb, out_rgb_ref),
                (x_ir_ref, g_ir_ref, inv_ir, s_ir, out_ir_ref)):
            picked = jnp.sum(x_ref[...] * g_ref[...], axis=1,
                             keepdims=True) * inv[...]
            out_ref[...] = jnp.mean(
                CAP + jnp.log(s_acc[...]) - picked).reshape(1, 1)


@jax.jit
def _run(x_rgb, x_ir, t_rgb, t_ir, f_rgb, f_ir):
    g_rgb, g_ir = _sc_gather(f_rgb, t_rgb, f_ir, t_ir)
    out = pl.pallas_call(
        _cm_kernel,
        grid=(N // TILE_N,),
        in_specs=[
            pl.BlockSpec((B, D), lambda c: (0, 0)),
            pl.BlockSpec((B, D), lambda c: (0, 0)),
            pl.BlockSpec((B, D), lambda c: (0, 0)),
            pl.BlockSpec((B, D), lambda c: (0, 0)),
            pl.BlockSpec((TILE_N, D), lambda c: (c, 0)),
            pl.BlockSpec((TILE_N, D), lambda c: (c, 0)),
        ],
        out_specs=[
            pl.BlockSpec((1, 1), lambda c: (0, 0)),
            pl.BlockSpec((1, 1), lambda c: (0, 0)),
        ],
        out_shape=[jax.ShapeDtypeStruct((1, 1), jnp.float32)] * 2,
        scratch_shapes=[
            pltpu.VMEM((B, D), jnp.bfloat16),
            pltpu.VMEM((B, D), jnp.bfloat16),
            pltpu.VMEM((B, 1), jnp.float32),
            pltpu.VMEM((B, 1), jnp.float32),
            pltpu.VMEM((B, 1), jnp.float32),
            pltpu.VMEM((B, 1), jnp.float32),
        ],
        compiler_params=pltpu.CompilerParams(
            dimension_semantics=("arbitrary",)),
    )(x_rgb, x_ir, g_rgb, g_ir, f_rgb, f_ir)
    return out[0][0, 0], out[1][0, 0]


def kernel(inputs_rgb, inputs_ir, targets_rgb, targets_ir,
           features_rgb, features_ir):
    return _run(inputs_rgb, inputs_ir, targets_rgb, targets_ir,
                features_rgb, features_ir)


# speedup vs baseline: 1.8990x; 1.0795x over previous
import functools
import math

import jax
import jax.numpy as jnp
from jax.experimental import pallas as pl
from jax.experimental.pallas import tpu as pltpu
from jax.experimental.pallas import tpu_sc as plsc

B = 1024
D = 128
N = 100000
TILE_N = 4000
TEMP = 0.05
LOG2E = math.log2(math.e)
QSCALE = LOG2E / TEMP

_info = plsc.get_sparse_core_info()
_NW = _info.num_cores * _info.num_subcores
_BPW = B // _NW


def _gather_body(f_rgb_hbm, t_rgb_hbm, f_ir_hbm, t_ir_hbm,
                 g_rgb_hbm, g_ir_hbm, idx_v, rows_v, sem):
    wid = (jax.lax.axis_index("s") * _info.num_cores
           + jax.lax.axis_index("c"))
    base = wid * _BPW
    for f_hbm, t_hbm, g_hbm in ((f_rgb_hbm, t_rgb_hbm, g_rgb_hbm),
                                (f_ir_hbm, t_ir_hbm, g_ir_hbm)):
        pltpu.sync_copy(t_hbm.at[pl.ds(base, _BPW)], idx_v)
        pltpu.async_copy(f_hbm.at[idx_v], rows_v, sem).wait()
        pltpu.sync_copy(rows_v, g_hbm.at[pl.ds(base, _BPW)])


_sc_gather = pl.kernel(
    _gather_body,
    mesh=plsc.VectorSubcoreMesh(core_axis_name="c", subcore_axis_name="s"),
    out_type=[jax.ShapeDtypeStruct((B, D), jnp.float32)] * 2,
    scratch_types=[
        pltpu.VMEM((_BPW,), jnp.int32),
        pltpu.VMEM((_BPW, D), jnp.float32),
        pltpu.SemaphoreType.DMA,
    ],
)


def _cm_kernel(x_rgb_ref, x_ir_ref, g_rgb_ref, g_ir_ref,
               f_rgb_ref, f_ir_ref, out_rgb_ref, out_ir_ref,
               xn_rgb, xn_ir, inv_rgb, inv_ir, s_rgb, s_ir):
    c = pl.program_id(0)
    nc = pl.num_programs(0)

    @pl.when(c == 0)
    def _init():
        for x_ref, xn, inv in ((x_rgb_ref, xn_rgb, inv_rgb),
                               (x_ir_ref, xn_ir, inv_ir)):
            x = x_ref[...]
            n = jnp.sqrt(jnp.sum(x * x, axis=1, keepdims=True))
            r = 1.0 / jnp.maximum(n, 1e-12)
            xn[...] = (x * (QSCALE * r)).astype(jnp.bfloat16)
            inv[...] = r * (1.0 / TEMP)
        s_rgb[...] = jnp.zeros_like(s_rgb)
        s_ir[...] = jnp.zeros_like(s_ir)

    def bank(xn, f_ref, s_acc):
        q = jax.lax.dot_general(
            xn[...], f_ref[...].astype(jnp.bfloat16),
            (((1,), (1,)), ((), ())),
            preferred_element_type=jnp.float32)
        s_acc[...] += jnp.sum(jnp.exp2(q), axis=1, keepdims=True)

    bank(xn_rgb, f_rgb_ref, s_rgb)
    bank(xn_ir, f_ir_ref, s_ir)

    @pl.when(c == nc - 1)
    def _fin():
        for x_ref, g_ref, inv, s_acc, out_ref in (
                (x_rgb_ref, g_rgb_ref, inv_rgb, s_rgb, out_rgb_ref),
                (x_ir_ref, g_ir_ref, inv_ir, s_ir, out_ir_ref)):
            picked = jnp.sum(x_ref[...] * g_ref[...], axis=1,
                             keepdims=True) * inv[...]
            out_ref[...] = jnp.mean(
                jnp.log(s_acc[...]) - picked).reshape(1, 1)


@jax.jit
def _run(x_rgb, x_ir, t_rgb, t_ir, f_rgb, f_ir):
    g_rgb, g_ir = _sc_gather(f_rgb, t_rgb, f_ir, t_ir)
    out = pl.pallas_call(
        _cm_kernel,
        grid=(N // TILE_N,),
        in_specs=[
            pl.BlockSpec((B, D), lambda c: (0, 0)),
            pl.BlockSpec((B, D), lambda c: (0, 0)),
            pl.BlockSpec((B, D), lambda c: (0, 0)),
            pl.BlockSpec((B, D), lambda c: (0, 0)),
            pl.BlockSpec((TILE_N, D), lambda c: (c, 0)),
            pl.BlockSpec((TILE_N, D), lambda c: (c, 0)),
        ],
        out_specs=[
            pl.BlockSpec((1, 1), lambda c: (0, 0)),
            pl.BlockSpec((1, 1), lambda c: (0, 0)),
        ],
        out_shape=[jax.ShapeDtypeStruct((1, 1), jnp.float32)] * 2,
        scratch_shapes=[
            pltpu.VMEM((B, D), jnp.bfloat16),
            pltpu.VMEM((B, D), jnp.bfloat16),
            pltpu.VMEM((B, 1), jnp.float32),
            pltpu.VMEM((B, 1), jnp.float32),
            pltpu.VMEM((B, 1), jnp.float32),
            pltpu.VMEM((B, 1), jnp.float32),
        ],
        compiler_params=pltpu.CompilerParams(
            dimension_semantics=("arbitrary",)),
    )(x_rgb, x_ir, g_rgb, g_ir, f_rgb, f_ir)
    return out[0][0, 0], out[1][0, 0]


def kernel(inputs_rgb, inputs_ir, targets_rgb, targets_ir,
           features_rgb, features_ir):
    return _run(inputs_rgb, inputs_ir, targets_rgb, targets_ir,
                features_rgb, features_ir)
